# Initial kernel scaffold; baseline (speedup 1.0000x reference)
import jax
import jax.numpy as jnp
from jax.experimental import pallas as pl


def kernel(x, params):
    raise NotImplementedError("write your pallas kernel here")


# speedup vs baseline: 1.6290x; 1.6290x over previous
import jax
import jax.numpy as jnp
from jax.experimental import pallas as pl
from jax.experimental.pallas import tpu as pltpu

_B = 2048
_FEAT = 128


def _body(x_ref, ws_ref, bs_ref, wc1_ref, wc2_ref, wc3_ref, bc_ref,
          w4_ref, b4_ref, wr_ref, br_ref, wpv_ref, bpv_ref,
          act_ref, lp_ref, ent_ref, val_ref):
    xb = x_ref[...]
    b = xb.shape[0]
    f32 = jnp.float32
    bf16 = jnp.bfloat16

    t0 = jax.nn.relu(xb[:, 7:8] * ws_ref[0:1, :] + bs_ref[0:1, :])
    t1 = jax.nn.relu(xb[:, 15:16] * ws_ref[1:2, :] + bs_ref[1:2, :])
    t5 = jax.nn.relu(xb[:, 47:48] * ws_ref[2:3, :] + bs_ref[2:3, :])

    c1 = jax.nn.relu(jnp.dot(xb[:, 16:24].astype(bf16), wc1_ref[...],
                             preferred_element_type=f32) + bc_ref[0:1, :])
    c2 = jax.nn.relu(jnp.dot(xb[:, 24:32].astype(bf16), wc2_ref[...],
                             preferred_element_type=f32) + bc_ref[1:2, :])
    c3 = jax.nn.relu(jnp.dot(xb[:, 32:40].astype(bf16), wc3_ref[...],
                             preferred_element_type=f32) + bc_ref[2:3, :])

    mh = []
    for h in range(8):
        sl = slice(h * _FEAT, (h + 1) * _FEAT)
        merged = jnp.concatenate(
            [t0[:, sl], t1[:, sl], c1[:, sl], c2[:, sl], c3[:, sl], t5[:, sl]],
            axis=1).astype(bf16)
        mh.append(jnp.dot(merged, w4_ref[h], preferred_element_type=f32)
                  + b4_ref[h:h + 1, :])

    zr = jnp.concatenate([mh[0], mh[1]], axis=1).astype(bf16)
    tr = jnp.dot(zr, wr_ref[...], preferred_element_type=f32) + br_ref[0:1, :]
    upds = []
    for r, base in ((0, 2), (1, 5)):
        logits = tr[:, r * 8:r * 8 + 3]
        nlog = tr[:, r * 8 + 3:r * 8 + 6]
        phase = tr[:, r * 8 + 6:r * 8 + 7]
        noise = jnp.sin(phase) * (jnp.maximum(nlog, 0.0)
                                  + jnp.log1p(jnp.exp(-jnp.abs(nlog))))
        noisy = logits + noise
        n0 = noisy[:, 0:1]
        n1 = noisy[:, 1:2]
        n2 = noisy[:, 2:3]
        e_idx = jnp.where(n0 >= n1,
                          jnp.where(n0 >= n2, 0, 2),
                          jnp.where(n1 >= n2, 1, 2)).astype(jnp.int32)
        upds.append(jnp.where(e_idx == 0, mh[base],
                              jnp.where(e_idx == 1, mh[base + 1],
                                        mh[base + 2])))

    zu = jnp.concatenate(upds, axis=1).astype(bf16)
    tu = jnp.dot(zu, wpv_ref[...], preferred_element_type=f32) + bpv_ref[0:1, :]
    la = tu[:, 0:6]
    value = tu[:, 8:9]

    lmax = jnp.max(la, axis=1, keepdims=True)
    sh = la - lmax
    ex = jnp.exp(sh)
    z = jnp.sum(ex, axis=1, keepdims=True)
    logp = sh - jnp.log(z)
    probs = ex / z
    entropy = -jnp.sum(probs * logp, axis=1, keepdims=True)
    iota = jax.lax.broadcasted_iota(jnp.int32, (b, 6), 1)
    action = jnp.min(jnp.where(la == lmax, iota, 6), axis=1, keepdims=True)
    log_prob = jnp.sum(jnp.where(iota == action, logp, 0.0), axis=1,
                       keepdims=True)

    act_ref[...] = action
    lp_ref[...] = log_prob
    ent_ref[...] = entropy
    val_ref[...] = value


def kernel(x, params):
    n = x.shape[0]
    f32 = jnp.float32
    bf16 = jnp.bfloat16
    x2d = x.reshape(n, 48)

    heads = [params["actor_router"], params["critic_router"],
             *params["actor_experts"], *params["critic_experts"]]

    ws = jnp.stack([jnp.concatenate([heads[h][name]["w"][:, 0]
                                     for h in range(8)])
                    for name in ("fc1", "fc2", "fc3")])
    bs = jnp.stack([jnp.concatenate([heads[h][name]["b"]
                                     for h in range(8)])
                    for name in ("fc1", "fc2", "fc3")])

    def conv_w(name, k):
        wt = jnp.concatenate([heads[h][name]["w"].T for h in range(8)], axis=1)
        if k < 8:
            wt = jnp.concatenate([wt, jnp.zeros((8 - k, 1024), f32)], axis=0)
        return wt.astype(bf16)

    wc1 = conv_w("conv1", 8)
    wc2 = conv_w("conv2", 8)
    wc3 = conv_w("conv3", 6)
    bc = jnp.stack([jnp.concatenate([heads[h][name]["b"] for h in range(8)])
                    for name in ("conv1", "conv2", "conv3")])

    w4 = jnp.stack([h["fc4"]["w"].T for h in heads]).astype(bf16)
    b4 = jnp.stack([h["fc4"]["b"] for h in heads])

    wr = jnp.zeros((256, 16), f32)
    br_vals = []
    for r in range(2):
        p = heads[r]
        blk = jnp.concatenate([p["topk"]["w"].T, p["noise"]["w"].T,
                               p["pseudo_proj"],
                               jnp.zeros((_FEAT, 1), f32)], axis=1)
        wr = wr.at[r * 128:(r + 1) * 128, r * 8:(r + 1) * 8].set(blk)
        br_vals.append(jnp.concatenate([p["topk"]["b"], p["noise"]["b"],
                                        jnp.zeros((2,), f32)]))
    wr = wr.astype(bf16)
    br = jnp.concatenate(br_vals)[None, :]

    wpv = jnp.zeros((256, 16), f32)
    wpv = wpv.at[0:128, 0:6].set(params["pi_head"]["w"].T)
    wpv = wpv.at[128:256, 8:9].set(params["val_head"]["w"].T)
    wpv = wpv.astype(bf16)
    bpv = jnp.zeros((16,), f32)
    bpv = bpv.at[0:6].set(params["pi_head"]["b"])
    bpv = bpv.at[8:9].set(params["val_head"]["b"])
    bpv = bpv[None, :]

    grid = (n // _B,)
    full = lambda shape: pl.BlockSpec(shape, lambda i: (0,) * len(shape))
    out_spec = pl.BlockSpec((_B, 1), lambda i: (i, 0))
    action, log_prob, entropy, value = pl.pallas_call(
        _body,
        grid=grid,
        in_specs=[
            pl.BlockSpec((_B, 48), lambda i: (i, 0)),
            full((3, 1024)), full((3, 1024)),
            full((8, 1024)), full((8, 1024)), full((8, 1024)),
            full((3, 1024)),
            full((8, 768, _FEAT)), full((8, _FEAT)),
            full((256, 16)), full((1, 16)),
            full((256, 16)), full((1, 16)),
        ],
        out_specs=[out_spec] * 4,
        out_shape=[
            jax.ShapeDtypeStruct((n, 1), jnp.int32),
            jax.ShapeDtypeStruct((n, 1), f32),
            jax.ShapeDtypeStruct((n, 1), f32),
            jax.ShapeDtypeStruct((n, 1), f32),
        ],
        compiler_params=pltpu.CompilerParams(
            dimension_semantics=("arbitrary",)),
    )(x2d, ws, bs, wc1, wc2, wc3, bc, w4, b4, wr, br, wpv, bpv)

    return (action.reshape(n), log_prob.reshape(n), entropy.reshape(n), value)


# speedup vs baseline: 2.3198x; 1.4241x over previous
import jax
import jax.numpy as jnp
from jax.experimental import pallas as pl
from jax.experimental.pallas import tpu as pltpu

_B = 2048
_FEAT = 128


def _body(x_ref, ws_ref, bs_ref, wc1_ref, wc2_ref, wc3_ref, bc_ref,
          w4_ref, b4_ref, wr_ref, br_ref, wpv_ref, bpv_ref,
          act_ref, lp_ref, ent_ref, val_ref):
    xb = x_ref[...]
    b = xb.shape[0]
    f32 = jnp.float32
    bf16 = jnp.bfloat16

    t0 = jax.nn.relu(xb[:, 7:8] * ws_ref[0:1, :] + bs_ref[0:1, :])
    t1 = jax.nn.relu(xb[:, 15:16] * ws_ref[1:2, :] + bs_ref[1:2, :])
    t5 = jax.nn.relu(xb[:, 47:48] * ws_ref[2:3, :] + bs_ref[2:3, :])

    c1 = jax.nn.relu(jnp.dot(xb[:, 16:24].astype(bf16), wc1_ref[...],
                             preferred_element_type=f32) + bc_ref[0:1, :])
    c2 = jax.nn.relu(jnp.dot(xb[:, 24:32].astype(bf16), wc2_ref[...],
                             preferred_element_type=f32) + bc_ref[1:2, :])
    c3 = jax.nn.relu(jnp.dot(xb[:, 32:40].astype(bf16), wc3_ref[...],
                             preferred_element_type=f32) + bc_ref[2:3, :])

    mh = []
    for h in range(8):
        sl = slice(h * _FEAT, (h + 1) * _FEAT)
        merged = jnp.concatenate(
            [t0[:, sl].astype(bf16), t1[:, sl].astype(bf16),
             c1[:, sl].astype(bf16), c2[:, sl].astype(bf16),
             c3[:, sl].astype(bf16), t5[:, sl].astype(bf16)], axis=1)
        mh.append(jnp.dot(merged, w4_ref[h], preferred_element_type=f32)
                  + b4_ref[h:h + 1, :])

    zr = jnp.concatenate([mh[0], mh[1]], axis=1).astype(bf16)
    tr = jnp.dot(zr, wr_ref[...], preferred_element_type=f32) + br_ref[0:1, :]
    trT = tr.T
    sinT = jnp.sin(trT)
    spT = jnp.maximum(trT, 0.0) + jnp.log1p(jnp.exp(-jnp.abs(trT)))
    eTs = []
    for r in (0, 1):
        logitsT = trT[r * 8:r * 8 + 3, :]
        noiseT = sinT[r * 8 + 6:r * 8 + 7, :] * spT[r * 8 + 3:r * 8 + 6, :]
        noisyT = logitsT + noiseT
        n0 = noisyT[0:1, :]
        n1 = noisyT[1:2, :]
        n2 = noisyT[2:3, :]
        eTs.append(jnp.where(n0 >= n1,
                             jnp.where(n0 >= n2, 0, 2),
                             jnp.where(n1 >= n2, 1, 2)).astype(jnp.int32))
    eB = jnp.concatenate(eTs, axis=0).T
    upds = []
    for r, base in ((0, 2), (1, 5)):
        e_idx = eB[:, r:r + 1]
        upds.append(jnp.where(e_idx == 0, mh[base],
                              jnp.where(e_idx == 1, mh[base + 1],
                                        mh[base + 2])))

    zu = jnp.concatenate(upds, axis=1).astype(bf16)
    tu = jnp.dot(zu, wpv_ref[...], preferred_element_type=f32) + bpv_ref[0:1, :]
    tuT = tu.T
    laT = tuT[0:6, :]
    valueT = tuT[8:9, :]

    lmaxT = jnp.max(laT, axis=0, keepdims=True)
    shT = laT - lmaxT
    exT = jnp.exp(shT)
    zT = jnp.sum(exT, axis=0, keepdims=True)
    logpT = shT - jnp.log(zT)
    probsT = exT / zT
    entropyT = -jnp.sum(probsT * logpT, axis=0, keepdims=True)
    iotaT = jax.lax.broadcasted_iota(jnp.int32, (6, b), 0)
    actionT = jnp.min(jnp.where(laT == lmaxT, iotaT, 6), axis=0, keepdims=True)
    log_probT = jnp.sum(jnp.where(iotaT == actionT, logpT, 0.0), axis=0,
                        keepdims=True)

    act_ref[...] = actionT[None]
    lp_ref[...] = log_probT[None]
    ent_ref[...] = entropyT[None]
    val_ref[...] = valueT[None]


def kernel(x, params):
    n = x.shape[0]
    f32 = jnp.float32
    bf16 = jnp.bfloat16
    x2d = x.reshape(n, 48)

    heads = [params["actor_router"], params["critic_router"],
             *params["actor_experts"], *params["critic_experts"]]

    ws = jnp.stack([jnp.concatenate([heads[h][name]["w"][:, 0]
                                     for h in range(8)])
                    for name in ("fc1", "fc2", "fc3")])
    bs = jnp.stack([jnp.concatenate([heads[h][name]["b"]
                                     for h in range(8)])
                    for name in ("fc1", "fc2", "fc3")])

    def conv_w(name, k):
        wt = jnp.concatenate([heads[h][name]["w"].T for h in range(8)], axis=1)
        if k < 8:
            wt = jnp.concatenate([wt, jnp.zeros((8 - k, 1024), f32)], axis=0)
        return wt.astype(bf16)

    wc1 = conv_w("conv1", 8)
    wc2 = conv_w("conv2", 8)
    wc3 = conv_w("conv3", 6)
    bc = jnp.stack([jnp.concatenate([heads[h][name]["b"] for h in range(8)])
                    for name in ("conv1", "conv2", "conv3")])

    w4 = jnp.stack([h["fc4"]["w"].T for h in heads]).astype(bf16)
    b4 = jnp.stack([h["fc4"]["b"] for h in heads])

    wr = jnp.zeros((256, 16), f32)
    br_vals = []
    for r in range(2):
        p = heads[r]
        blk = jnp.concatenate([p["topk"]["w"].T, p["noise"]["w"].T,
                               p["pseudo_proj"],
                               jnp.zeros((_FEAT, 1), f32)], axis=1)
        wr = wr.at[r * 128:(r + 1) * 128, r * 8:(r + 1) * 8].set(blk)
        br_vals.append(jnp.concatenate([p["topk"]["b"], p["noise"]["b"],
                                        jnp.zeros((2,), f32)]))
    wr = wr.astype(bf16)
    br = jnp.concatenate(br_vals)[None, :]

    wpv = jnp.zeros((256, 16), f32)
    wpv = wpv.at[0:128, 0:6].set(params["pi_head"]["w"].T)
    wpv = wpv.at[128:256, 8:9].set(params["val_head"]["w"].T)
    wpv = wpv.astype(bf16)
    bpv = jnp.zeros((16,), f32)
    bpv = bpv.at[0:6].set(params["pi_head"]["b"])
    bpv = bpv.at[8:9].set(params["val_head"]["b"])
    bpv = bpv[None, :]

    grid = (n // _B,)
    full = lambda shape: pl.BlockSpec(shape, lambda i: (0,) * len(shape))
    out_spec = pl.BlockSpec((1, 1, _B), lambda i: (i, 0, 0))
    action, log_prob, entropy, value = pl.pallas_call(
        _body,
        grid=grid,
        in_specs=[
            pl.BlockSpec((_B, 48), lambda i: (i, 0)),
            full((3, 1024)), full((3, 1024)),
            full((8, 1024)), full((8, 1024)), full((8, 1024)),
            full((3, 1024)),
            full((8, 768, _FEAT)), full((8, _FEAT)),
            full((256, 16)), full((1, 16)),
            full((256, 16)), full((1, 16)),
        ],
        out_specs=[out_spec] * 4,
        out_shape=[
            jax.ShapeDtypeStruct((n // _B, 1, _B), jnp.int32),
            jax.ShapeDtypeStruct((n // _B, 1, _B), f32),
            jax.ShapeDtypeStruct((n // _B, 1, _B), f32),
            jax.ShapeDtypeStruct((n // _B, 1, _B), f32),
        ],
        compiler_params=pltpu.CompilerParams(
            dimension_semantics=("arbitrary",)),
    )(x2d, ws, bs, wc1, wc2, wc3, bc, w4, b4, wr, br, wpv, bpv)

    return (action.reshape(n), log_prob.reshape(n), entropy.reshape(n),
            value.reshape(n, 1))
